# baseline (speedup 1.0000x reference)
import functools

import jax
import jax.numpy as jnp
from jax import lax
from jax.experimental import pallas as pl
from jax.experimental.pallas import tpu as pltpu
from jax.experimental.pallas import tpu_sc as plsc

VOCAB = 1000000
EMBED_DIM = 64
BATCH = 4096
TLEN = 200

_info = plsc.get_sparse_core_info()
NC, NS = _info.num_cores, _info.num_subcores
NW = NC * NS
ROWS_PER_W = BATCH // NW
N_CHUNK = ROWS_PER_W
NBUF = 4

_mesh = plsc.VectorSubcoreMesh(core_axis_name="c", subcore_axis_name="s")


@functools.partial(
    pl.kernel,
    mesh=_mesh,
    out_type=jax.ShapeDtypeStruct((BATCH, TLEN, EMBED_DIM), jnp.float32),
    scratch_types=[
        pltpu.VMEM((ROWS_PER_W, TLEN), jnp.int32),
        pltpu.VMEM((NBUF, TLEN, EMBED_DIM), jnp.float32),
        pltpu.SemaphoreType.DMA((NBUF,)),
        pltpu.SemaphoreType.DMA((NBUF,)),
    ],
    compiler_params=pltpu.CompilerParams(use_tc_tiling_on_sc=False),
)
def _embed_gather(idx_hbm, table_hbm, out_hbm, idx_v, rows_v, gsem, ssem):
    wid = lax.axis_index("s") * NC + lax.axis_index("c")
    row0 = wid * ROWS_PER_W
    pltpu.sync_copy(idx_hbm.at[pl.ds(row0, ROWS_PER_W)], idx_v)

    def gather(i, b):
        return pltpu.make_async_copy(
            table_hbm.at[idx_v.at[i]],
            rows_v.at[b],
            gsem.at[b],
        )

    def store(i, b):
        return pltpu.make_async_copy(
            rows_v.at[b],
            out_hbm.at[row0 + i],
            ssem.at[b],
        )

    for b in range(NBUF):
        gather(b, b).start()

    @pl.loop(0, N_CHUNK - NBUF, step=NBUF)
    def _(i0):
        for b in range(NBUF):
            i = i0 + b
            gather(i, b).wait()
            store(i, b).start()
            store(i, b).wait()
            gather(i + NBUF, b).start()

    for b in range(NBUF):
        i = N_CHUNK - NBUF + b
        gather(i, b).wait()
        store(i, b).start()
        store(i, b).wait()


def kernel(x, table):
    out = _embed_gather(x.astype(jnp.int32), table)
    scale = (x[0, 0] * 0 + 1).astype(jnp.float32)
    return out * scale


# speedup vs baseline: 1.0025x; 1.0025x over previous
import functools

import jax
import jax.numpy as jnp
from jax import lax
from jax.experimental import pallas as pl
from jax.experimental.pallas import tpu as pltpu
from jax.experimental.pallas import tpu_sc as plsc

VOCAB = 1000000
EMBED_DIM = 64
BATCH = 4096
TLEN = 200

_info = plsc.get_sparse_core_info()
NC, NS = _info.num_cores, _info.num_subcores
NW = NC * NS
ROWS_PER_W = BATCH // NW
N_CHUNK = ROWS_PER_W
NBUF = 4

_mesh = plsc.VectorSubcoreMesh(core_axis_name="c", subcore_axis_name="s")


@functools.partial(
    pl.kernel,
    mesh=_mesh,
    out_type=jax.ShapeDtypeStruct((BATCH, TLEN, EMBED_DIM), jnp.float32),
    scratch_types=[
        pltpu.VMEM((ROWS_PER_W, TLEN), jnp.int32),
        pltpu.VMEM((NBUF, TLEN, EMBED_DIM), jnp.float32),
        pltpu.SemaphoreType.DMA((NBUF,)),
        pltpu.SemaphoreType.DMA((NBUF,)),
    ],
    compiler_params=pltpu.CompilerParams(
        use_tc_tiling_on_sc=False,
        skip_device_barrier=True,
    ),
)
def _embed_gather(idx_hbm, table_hbm, out_hbm, idx_v, rows_v, gsem, ssem):
    wid = lax.axis_index("s") * NC + lax.axis_index("c")
    row0 = wid * ROWS_PER_W
    pltpu.sync_copy(idx_hbm.at[pl.ds(row0, ROWS_PER_W)], idx_v)

    def gather(i, b):
        return pltpu.make_async_copy(
            table_hbm.at[idx_v.at[i]],
            rows_v.at[b],
            gsem.at[b],
        )

    def store(i, b):
        return pltpu.make_async_copy(
            rows_v.at[b],
            out_hbm.at[row0 + i],
            ssem.at[b],
        )

    for b in range(NBUF):
        gather(b, b).start()

    @pl.loop(0, N_CHUNK - NBUF, step=NBUF)
    def _(i0):
        for b in range(NBUF):
            i = i0 + b
            gather(i, b).wait()
            store(i, b).start()
            store(i, b).wait()
            gather(i + NBUF, b).start()

    for b in range(NBUF):
        i = N_CHUNK - NBUF + b
        gather(i, b).wait()
        store(i, b).start()
        store(i, b).wait()


def kernel(x, table):
    return _embed_gather(x.astype(jnp.int32), table)
